# baseline (speedup 1.0000x reference)
import jax
import jax.numpy as jnp
from jax import lax
from jax.experimental import pallas as pl
from jax.experimental.pallas import tpu as pltpu
from jax.experimental.pallas import tpu_sc as plsc

N_NODES = 10000
N_EDGES = 320000
D = 128
N_GRAPHS = 64

NC = 2
NS = 16
N_TILES = NC * NS
CHUNK = 128
E_PAD = 327680
EDGE_ROWS = E_PAD // CHUNK
N_PAD = 10112
FPT = D // N_TILES
EBLK = 16
NBLK = EDGE_ROWS // EBLK

_HI = lax.Precision.HIGHEST


def _sc_agg_body(vt_hbm, src_hbm, dst_hbm, w_hbm, zero_hbm, out_hbm,
                 idx_v, ia_v, ib_v,
                 yt0, yt1, yt2, yt3, at0, at1, at2, at3,
                 sa_v, da_v, wa_v, sb_v, db_v, wb_v,
                 sem, sem_a, sem_b):
    cid = lax.axis_index("c")
    sid = lax.axis_index("s")
    wid = cid * NS + sid
    yts = [yt0, yt1, yt2, yt3]
    ats = [at0, at1, at2, at3]
    for f in range(FPT):
        idx_v[...] = lax.iota(jnp.int32, 16) + (wid * FPT + f)
        pltpu.async_copy(vt_hbm.at[idx_v.at[pl.ds(0, 1)]], yts[f], sem).wait()
        pltpu.sync_copy(zero_hbm, ats[f])

    def issue(bi, i_ref, s_ref, d_ref, w_ref, s_sem):
        for t in range(EBLK // 16):
            i_ref[pl.ds(t * 16, 16)] = lax.iota(jnp.int32, 16) + (bi * EBLK + t * 16)
        pltpu.async_copy(src_hbm.at[i_ref], s_ref, s_sem)
        pltpu.async_copy(dst_hbm.at[i_ref], d_ref, s_sem)
        pltpu.async_copy(w_hbm.at[i_ref], w_ref, s_sem)

    def wait3(s_ref, d_ref, w_ref, s_sem):
        pltpu.make_async_copy(src_hbm.at[ia_v], s_ref, s_sem).wait()
        pltpu.make_async_copy(dst_hbm.at[ia_v], d_ref, s_sem).wait()
        pltpu.make_async_copy(w_hbm.at[ia_v], w_ref, s_sem).wait()

    def compute(s_ref, d_ref, w_ref):
        @plsc.parallel_loop(0, EBLK, 1, unroll=8)
        def row_body(r):
            for j in range(CHUNK // 16):
                s16 = s_ref[r, pl.ds(j * 16, 16)]
                d16 = d_ref[r, pl.ds(j * 16, 16)]
                w16 = w_ref[r, pl.ds(j * 16, 16)]
                for f in range(FPT):
                    v = plsc.load_gather(yts[f].at[0], [s16])
                    plsc.addupdate_scatter(ats[f].at[0], [d16], v * w16)

    issue(0, ia_v, sa_v, da_v, wa_v, sem_a)

    def pair_body(k, carry):
        ba = 2 * k
        wait3(sa_v, da_v, wa_v, sem_a)
        issue(ba + 1, ib_v, sb_v, db_v, wb_v, sem_b)
        compute(sa_v, da_v, wa_v)
        wait3(sb_v, db_v, wb_v, sem_b)

        @pl.when(k < NBLK // 2 - 1)
        def _():
            issue(ba + 2, ia_v, sa_v, da_v, wa_v, sem_a)

        compute(sb_v, db_v, wb_v)
        return carry

    lax.fori_loop(0, NBLK // 2, pair_body, 0)
    for f in range(FPT):
        pltpu.sync_copy(ats[f], out_hbm.at[wid * FPT + f])


def _sc_agg(vt, src2, dst2, w2, zeros_row, mesh, sc_params):
    out = pl.kernel(
        _sc_agg_body,
        out_type=jax.ShapeDtypeStruct((D, 1, N_PAD), jnp.float32),
        mesh=mesh,
        compiler_params=sc_params,
        scratch_types=[
            pltpu.VMEM((16,), jnp.int32),
            pltpu.VMEM((EBLK,), jnp.int32),
            pltpu.VMEM((EBLK,), jnp.int32),
            pltpu.VMEM((1, N_PAD), jnp.float32),
            pltpu.VMEM((1, N_PAD), jnp.float32),
            pltpu.VMEM((1, N_PAD), jnp.float32),
            pltpu.VMEM((1, N_PAD), jnp.float32),
            pltpu.VMEM((1, N_PAD), jnp.float32),
            pltpu.VMEM((1, N_PAD), jnp.float32),
            pltpu.VMEM((1, N_PAD), jnp.float32),
            pltpu.VMEM((1, N_PAD), jnp.float32),
            pltpu.VMEM((EBLK, CHUNK), jnp.int32),
            pltpu.VMEM((EBLK, CHUNK), jnp.int32),
            pltpu.VMEM((EBLK, CHUNK), jnp.float32),
            pltpu.VMEM((EBLK, CHUNK), jnp.int32),
            pltpu.VMEM((EBLK, CHUNK), jnp.int32),
            pltpu.VMEM((EBLK, CHUNK), jnp.float32),
            pltpu.SemaphoreType.DMA,
            pltpu.SemaphoreType.DMA,
            pltpu.SemaphoreType.DMA,
        ],
    )(vt, src2, dst2, w2, zeros_row)
    return out.reshape(D, N_PAD)


def _mid_body(agg_ref, xt_ref, wr1_ref, wt1_ref, br1_ref, out_ref):
    dn = (((1,), (0,)), ((), ()))
    pre = lax.dot_general(wr1_ref[...], agg_ref[...], dn,
                          preferred_element_type=jnp.float32) \
        + lax.dot_general(wt1_ref[...], xt_ref[...], dn,
                          preferred_element_type=jnp.float32) \
        + br1_ref[...]
    out_ref[...] = jnp.maximum(pre, 0.0)


def _post_body(agg_ref, ht_ref, wr3_ref, wt3_ref, br3_ref, batch_ref,
               wlin_ref, blin_ref, out_ref):
    dn = (((1,), (0,)), ((), ()))
    h2 = lax.dot_general(wr3_ref[...], agg_ref[...], dn,
                         preferred_element_type=jnp.float32) \
        + lax.dot_general(wt3_ref[...], ht_ref[...], dn,
                          preferred_element_type=jnp.float32) \
        + br3_ref[...]
    b = batch_ref[...]
    gids = lax.broadcasted_iota(jnp.int32, (N_GRAPHS, N_PAD), 0)
    oh = jnp.where(gids == b, 1.0, 0.0).astype(jnp.float32)
    dn1 = (((1,), (1,)), ((), ()))
    sums = lax.dot_general(h2, oh, dn1, preferred_element_type=jnp.float32,
                           precision=_HI)
    counts = lax.dot_general(jnp.ones((1, N_PAD), jnp.float32), oh, dn1,
                             preferred_element_type=jnp.float32,
                             precision=_HI)
    g = sums / jnp.maximum(counts, 1.0)
    res = lax.dot_general(wlin_ref[...], g, dn,
                          preferred_element_type=jnp.float32) + blin_ref[...]
    out_ref[...] = jnp.broadcast_to(jnp.maximum(res, 0.0), (8, N_GRAPHS))


def kernel(x, edge_index, batch, edge_attr, W_rel1, b_rel1, W_root1,
           W_rel3, b_rel3, W_root3, W_lin, b_lin):
    f32 = jnp.float32
    pad = E_PAD - N_EDGES
    src2 = jnp.concatenate(
        [edge_index[0].astype(jnp.int32), jnp.zeros((pad,), jnp.int32)]
    ).reshape(EDGE_ROWS, CHUNK)
    dst2 = jnp.concatenate(
        [edge_index[1].astype(jnp.int32), jnp.zeros((pad,), jnp.int32)]
    ).reshape(EDGE_ROWS, CHUNK)
    w2 = jnp.concatenate(
        [edge_attr.astype(f32), jnp.zeros((pad,), f32)]
    ).reshape(EDGE_ROWS, CHUNK)
    xt = jnp.pad(x.T, ((0, 0), (0, N_PAD - N_NODES)))
    batch_p = jnp.pad(batch.astype(jnp.int32), (0, N_PAD - N_NODES),
                      constant_values=N_GRAPHS).reshape(1, N_PAD)
    zeros_row = jnp.zeros((1, N_PAD), f32)

    mesh = plsc.VectorSubcoreMesh(core_axis_name="c", subcore_axis_name="s")
    sc_params = pltpu.CompilerParams(needs_layout_passes=False)

    agg1t = _sc_agg(xt, src2, dst2, w2, zeros_row, mesh, sc_params)

    h1t = pl.pallas_call(
        _mid_body,
        out_shape=jax.ShapeDtypeStruct((D, N_PAD), f32),
    )(agg1t, xt, W_rel1, W_root1, b_rel1.reshape(D, 1))

    agg3t = _sc_agg(h1t, src2, dst2, w2, zeros_row, mesh, sc_params)

    pooled = pl.pallas_call(
        _post_body,
        out_shape=jax.ShapeDtypeStruct((8, N_GRAPHS), f32),
    )(agg3t, h1t, W_rel3, W_root3, b_rel3.reshape(D, 1), batch_p,
      W_lin, b_lin.reshape(1, 1))

    return pooled[0].reshape(N_GRAPHS, 1)


# speedup vs baseline: 1.1586x; 1.1586x over previous
import jax
import jax.numpy as jnp
from jax import lax
from jax.experimental import pallas as pl
from jax.experimental.pallas import tpu as pltpu
from jax.experimental.pallas import tpu_sc as plsc

N_NODES = 10000
N_EDGES = 320000
D = 128
N_GRAPHS = 64

NC = 2
NS = 16
N_TILES = NC * NS
CHUNK = 128
E_PAD = 327680
EDGE_ROWS = E_PAD // CHUNK
N_PAD = 10112
FPT = D // N_TILES
EBLK = 16
NBLK = EDGE_ROWS // EBLK

_HI = lax.Precision.HIGHEST


def _sc_agg_body(vt_hbm, src_hbm, dst_hbm, w_hbm, zero_hbm, out_hbm,
                 idx_v, ia_v, ib_v,
                 yt0, yt1, yt2, yt3, at0, at1, at2, at3,
                 sa_v, da_v, wa_v, sb_v, db_v, wb_v,
                 sem, sem_a, sem_b):
    cid = lax.axis_index("c")
    sid = lax.axis_index("s")
    wid = cid * NS + sid
    yts = [yt0, yt1, yt2, yt3]
    ats = [at0, at1, at2, at3]
    for f in range(FPT):
        idx_v[...] = lax.iota(jnp.int32, 16) + (wid * FPT + f)
        pltpu.async_copy(vt_hbm.at[idx_v.at[pl.ds(0, 1)]], yts[f], sem).wait()
        pltpu.sync_copy(zero_hbm, ats[f])

    def issue(bi, i_ref, s_ref, d_ref, w_ref, s_sem):
        for t in range(EBLK // 16):
            i_ref[pl.ds(t * 16, 16)] = lax.iota(jnp.int32, 16) + (bi * EBLK + t * 16)
        pltpu.async_copy(src_hbm.at[i_ref], s_ref, s_sem)
        pltpu.async_copy(dst_hbm.at[i_ref], d_ref, s_sem)
        pltpu.async_copy(w_hbm.at[i_ref], w_ref, s_sem)

    def wait3(s_ref, d_ref, w_ref, s_sem):
        pltpu.make_async_copy(src_hbm.at[ia_v], s_ref, s_sem).wait()
        pltpu.make_async_copy(dst_hbm.at[ia_v], d_ref, s_sem).wait()
        pltpu.make_async_copy(w_hbm.at[ia_v], w_ref, s_sem).wait()

    def compute(s_ref, d_ref, w_ref):
        @plsc.parallel_loop(0, EBLK, 1, unroll=4)
        def row_body(r):
            for j in range(CHUNK // 16):
                s16 = s_ref[r, pl.ds(j * 16, 16)]
                d16 = d_ref[r, pl.ds(j * 16, 16)]
                w16 = w_ref[r, pl.ds(j * 16, 16)]
                for f in range(FPT):
                    v = plsc.load_gather(yts[f].at[0], [s16])
                    plsc.addupdate_scatter(ats[f].at[0], [d16], v * w16)

    issue(0, ia_v, sa_v, da_v, wa_v, sem_a)

    def pair_body(k, carry):
        ba = 2 * k
        wait3(sa_v, da_v, wa_v, sem_a)
        issue(ba + 1, ib_v, sb_v, db_v, wb_v, sem_b)
        compute(sa_v, da_v, wa_v)
        wait3(sb_v, db_v, wb_v, sem_b)

        @pl.when(k < NBLK // 2 - 1)
        def _():
            issue(ba + 2, ia_v, sa_v, da_v, wa_v, sem_a)

        compute(sb_v, db_v, wb_v)
        return carry

    lax.fori_loop(0, NBLK // 2, pair_body, 0)
    for f in range(FPT):
        pltpu.sync_copy(ats[f], out_hbm.at[wid * FPT + f])


def _sc_agg(vt, src2, dst2, w2, zeros_row, mesh, sc_params):
    out = pl.kernel(
        _sc_agg_body,
        out_type=jax.ShapeDtypeStruct((D, 1, N_PAD), jnp.float32),
        mesh=mesh,
        compiler_params=sc_params,
        scratch_types=[
            pltpu.VMEM((16,), jnp.int32),
            pltpu.VMEM((EBLK,), jnp.int32),
            pltpu.VMEM((EBLK,), jnp.int32),
            pltpu.VMEM((1, N_PAD), jnp.float32),
            pltpu.VMEM((1, N_PAD), jnp.float32),
            pltpu.VMEM((1, N_PAD), jnp.float32),
            pltpu.VMEM((1, N_PAD), jnp.float32),
            pltpu.VMEM((1, N_PAD), jnp.float32),
            pltpu.VMEM((1, N_PAD), jnp.float32),
            pltpu.VMEM((1, N_PAD), jnp.float32),
            pltpu.VMEM((1, N_PAD), jnp.float32),
            pltpu.VMEM((EBLK, CHUNK), jnp.int32),
            pltpu.VMEM((EBLK, CHUNK), jnp.int32),
            pltpu.VMEM((EBLK, CHUNK), jnp.float32),
            pltpu.VMEM((EBLK, CHUNK), jnp.int32),
            pltpu.VMEM((EBLK, CHUNK), jnp.int32),
            pltpu.VMEM((EBLK, CHUNK), jnp.float32),
            pltpu.SemaphoreType.DMA,
            pltpu.SemaphoreType.DMA,
            pltpu.SemaphoreType.DMA,
        ],
    )(vt, src2, dst2, w2, zeros_row)
    return out.reshape(D, N_PAD)


def _mid_body(agg_ref, xt_ref, wr1_ref, wt1_ref, br1_ref, out_ref):
    dn = (((1,), (0,)), ((), ()))
    pre = lax.dot_general(wr1_ref[...], agg_ref[...], dn,
                          preferred_element_type=jnp.float32) \
        + lax.dot_general(wt1_ref[...], xt_ref[...], dn,
                          preferred_element_type=jnp.float32) \
        + br1_ref[...]
    out_ref[...] = jnp.maximum(pre, 0.0)


def _post_body(agg_ref, ht_ref, wr3_ref, wt3_ref, br3_ref, batch_ref,
               wlin_ref, blin_ref, out_ref):
    dn = (((1,), (0,)), ((), ()))
    h2 = lax.dot_general(wr3_ref[...], agg_ref[...], dn,
                         preferred_element_type=jnp.float32) \
        + lax.dot_general(wt3_ref[...], ht_ref[...], dn,
                          preferred_element_type=jnp.float32) \
        + br3_ref[...]
    b = batch_ref[...]
    gids = lax.broadcasted_iota(jnp.int32, (N_GRAPHS, N_PAD), 0)
    oh = jnp.where(gids == b, 1.0, 0.0).astype(jnp.float32)
    dn1 = (((1,), (1,)), ((), ()))
    sums = lax.dot_general(h2, oh, dn1, preferred_element_type=jnp.float32,
                           precision=_HI)
    counts = lax.dot_general(jnp.ones((1, N_PAD), jnp.float32), oh, dn1,
                             preferred_element_type=jnp.float32,
                             precision=_HI)
    g = sums / jnp.maximum(counts, 1.0)
    res = lax.dot_general(wlin_ref[...], g, dn,
                          preferred_element_type=jnp.float32) + blin_ref[...]
    out_ref[...] = jnp.broadcast_to(jnp.maximum(res, 0.0), (8, N_GRAPHS))


def kernel(x, edge_index, batch, edge_attr, W_rel1, b_rel1, W_root1,
           W_rel3, b_rel3, W_root3, W_lin, b_lin):
    f32 = jnp.float32
    pad = E_PAD - N_EDGES
    src2 = jnp.concatenate(
        [edge_index[0].astype(jnp.int32), jnp.zeros((pad,), jnp.int32)]
    ).reshape(EDGE_ROWS, CHUNK)
    dst2 = jnp.concatenate(
        [edge_index[1].astype(jnp.int32), jnp.zeros((pad,), jnp.int32)]
    ).reshape(EDGE_ROWS, CHUNK)
    w2 = jnp.concatenate(
        [edge_attr.astype(f32), jnp.zeros((pad,), f32)]
    ).reshape(EDGE_ROWS, CHUNK)
    xt = jnp.pad(x.T, ((0, 0), (0, N_PAD - N_NODES)))
    batch_p = jnp.pad(batch.astype(jnp.int32), (0, N_PAD - N_NODES),
                      constant_values=N_GRAPHS).reshape(1, N_PAD)
    zeros_row = jnp.zeros((1, N_PAD), f32)

    mesh = plsc.VectorSubcoreMesh(core_axis_name="c", subcore_axis_name="s")
    sc_params = pltpu.CompilerParams(needs_layout_passes=False)

    agg1t = _sc_agg(xt, src2, dst2, w2, zeros_row, mesh, sc_params)

    h1t = pl.pallas_call(
        _mid_body,
        out_shape=jax.ShapeDtypeStruct((D, N_PAD), f32),
    )(agg1t, xt, W_rel1, W_root1, b_rel1.reshape(D, 1))

    agg3t = _sc_agg(h1t, src2, dst2, w2, zeros_row, mesh, sc_params)

    pooled = pl.pallas_call(
        _post_body,
        out_shape=jax.ShapeDtypeStruct((8, N_GRAPHS), f32),
    )(agg3t, h1t, W_rel3, W_root3, b_rel3.reshape(D, 1), batch_p,
      W_lin, b_lin.reshape(1, 1))

    return pooled[0].reshape(N_GRAPHS, 1)


# speedup vs baseline: 1.3872x; 1.1973x over previous
import jax
import jax.numpy as jnp
from jax import lax
from jax.experimental import pallas as pl
from jax.experimental.pallas import tpu as pltpu
from jax.experimental.pallas import tpu_sc as plsc

N_NODES = 10000
N_EDGES = 320000
D = 128
N_GRAPHS = 64

NC = 2
NS = 16
N_TILES = NC * NS
CHUNK = 128
E_PAD = 327680
EDGE_ROWS = E_PAD // CHUNK
N_PAD = 10112
FPT = D // N_TILES
EBLK = 32
NBLK = EDGE_ROWS // EBLK

_HI = lax.Precision.HIGHEST


def _sc_agg_body(vt_hbm, src_hbm, dst_hbm, w_hbm, zero_hbm, out_hbm,
                 idx_v, ia_v, ib_v,
                 yt0, yt1, yt2, yt3, at0, at1, at2, at3,
                 sa_v, da_v, wa_v, sb_v, db_v, wb_v,
                 sem, sem_a, sem_b):
    cid = lax.axis_index("c")
    sid = lax.axis_index("s")
    wid = cid * NS + sid
    yts = [yt0, yt1, yt2, yt3]
    ats = [at0, at1, at2, at3]
    for f in range(FPT):
        idx_v[...] = lax.iota(jnp.int32, 16) + (wid * FPT + f)
        pltpu.async_copy(vt_hbm.at[idx_v.at[pl.ds(0, 1)]], yts[f], sem).wait()
        pltpu.sync_copy(zero_hbm, ats[f])

    def issue(bi, i_ref, s_ref, d_ref, w_ref, s_sem):
        for t in range(EBLK // 16):
            i_ref[pl.ds(t * 16, 16)] = lax.iota(jnp.int32, 16) + (bi * EBLK + t * 16)
        pltpu.async_copy(src_hbm.at[i_ref], s_ref, s_sem)
        pltpu.async_copy(dst_hbm.at[i_ref], d_ref, s_sem)
        pltpu.async_copy(w_hbm.at[i_ref], w_ref, s_sem)

    def wait3(s_ref, d_ref, w_ref, s_sem):
        pltpu.make_async_copy(src_hbm.at[ia_v], s_ref, s_sem).wait()
        pltpu.make_async_copy(dst_hbm.at[ia_v], d_ref, s_sem).wait()
        pltpu.make_async_copy(w_hbm.at[ia_v], w_ref, s_sem).wait()

    def compute(s_ref, d_ref, w_ref):
        @plsc.parallel_loop(0, EBLK, 1, unroll=4)
        def row_body(r):
            for j in range(CHUNK // 16):
                s16 = s_ref[r, pl.ds(j * 16, 16)]
                d16 = d_ref[r, pl.ds(j * 16, 16)]
                w16 = w_ref[r, pl.ds(j * 16, 16)]
                for f in range(FPT):
                    v = plsc.load_gather(yts[f].at[0], [s16])
                    plsc.addupdate_scatter(ats[f].at[0], [d16], v * w16)

    issue(0, ia_v, sa_v, da_v, wa_v, sem_a)

    def pair_body(k, carry):
        ba = 2 * k
        wait3(sa_v, da_v, wa_v, sem_a)
        issue(ba + 1, ib_v, sb_v, db_v, wb_v, sem_b)
        compute(sa_v, da_v, wa_v)
        wait3(sb_v, db_v, wb_v, sem_b)

        @pl.when(k < NBLK // 2 - 1)
        def _():
            issue(ba + 2, ia_v, sa_v, da_v, wa_v, sem_a)

        compute(sb_v, db_v, wb_v)
        return carry

    lax.fori_loop(0, NBLK // 2, pair_body, 0)
    for f in range(FPT):
        pltpu.sync_copy(ats[f], out_hbm.at[wid * FPT + f])


def _sc_agg(vt, src2, dst2, w2, zeros_row, mesh, sc_params):
    out = pl.kernel(
        _sc_agg_body,
        out_type=jax.ShapeDtypeStruct((D, 1, N_PAD), jnp.float32),
        mesh=mesh,
        compiler_params=sc_params,
        scratch_types=[
            pltpu.VMEM((16,), jnp.int32),
            pltpu.VMEM((EBLK,), jnp.int32),
            pltpu.VMEM((EBLK,), jnp.int32),
            pltpu.VMEM((1, N_PAD), jnp.float32),
            pltpu.VMEM((1, N_PAD), jnp.float32),
            pltpu.VMEM((1, N_PAD), jnp.float32),
            pltpu.VMEM((1, N_PAD), jnp.float32),
            pltpu.VMEM((1, N_PAD), jnp.float32),
            pltpu.VMEM((1, N_PAD), jnp.float32),
            pltpu.VMEM((1, N_PAD), jnp.float32),
            pltpu.VMEM((1, N_PAD), jnp.float32),
            pltpu.VMEM((EBLK, CHUNK), jnp.int32),
            pltpu.VMEM((EBLK, CHUNK), jnp.int32),
            pltpu.VMEM((EBLK, CHUNK), jnp.float32),
            pltpu.VMEM((EBLK, CHUNK), jnp.int32),
            pltpu.VMEM((EBLK, CHUNK), jnp.int32),
            pltpu.VMEM((EBLK, CHUNK), jnp.float32),
            pltpu.SemaphoreType.DMA,
            pltpu.SemaphoreType.DMA,
            pltpu.SemaphoreType.DMA,
        ],
    )(vt, src2, dst2, w2, zeros_row)
    return out.reshape(D, N_PAD)


def _mid_body(agg_ref, xt_ref, wr1_ref, wt1_ref, br1_ref, out_ref):
    dn = (((1,), (0,)), ((), ()))
    pre = lax.dot_general(wr1_ref[...], agg_ref[...], dn,
                          preferred_element_type=jnp.float32) \
        + lax.dot_general(wt1_ref[...], xt_ref[...], dn,
                          preferred_element_type=jnp.float32) \
        + br1_ref[...]
    out_ref[...] = jnp.maximum(pre, 0.0)


def _post_body(agg_ref, ht_ref, wr3_ref, wt3_ref, br3_ref, batch_ref,
               wlin_ref, blin_ref, out_ref):
    dn = (((1,), (0,)), ((), ()))
    h2 = lax.dot_general(wr3_ref[...], agg_ref[...], dn,
                         preferred_element_type=jnp.float32) \
        + lax.dot_general(wt3_ref[...], ht_ref[...], dn,
                          preferred_element_type=jnp.float32) \
        + br3_ref[...]
    b = batch_ref[...]
    gids = lax.broadcasted_iota(jnp.int32, (N_GRAPHS, N_PAD), 0)
    oh = jnp.where(gids == b, 1.0, 0.0).astype(jnp.float32)
    dn1 = (((1,), (1,)), ((), ()))
    sums = lax.dot_general(h2, oh, dn1, preferred_element_type=jnp.float32,
                           precision=_HI)
    counts = lax.dot_general(jnp.ones((1, N_PAD), jnp.float32), oh, dn1,
                             preferred_element_type=jnp.float32,
                             precision=_HI)
    g = sums / jnp.maximum(counts, 1.0)
    res = lax.dot_general(wlin_ref[...], g, dn,
                          preferred_element_type=jnp.float32) + blin_ref[...]
    out_ref[...] = jnp.broadcast_to(jnp.maximum(res, 0.0), (8, N_GRAPHS))


def kernel(x, edge_index, batch, edge_attr, W_rel1, b_rel1, W_root1,
           W_rel3, b_rel3, W_root3, W_lin, b_lin):
    f32 = jnp.float32
    pad = E_PAD - N_EDGES
    src2 = jnp.concatenate(
        [edge_index[0].astype(jnp.int32), jnp.zeros((pad,), jnp.int32)]
    ).reshape(EDGE_ROWS, CHUNK)
    dst2 = jnp.concatenate(
        [edge_index[1].astype(jnp.int32), jnp.zeros((pad,), jnp.int32)]
    ).reshape(EDGE_ROWS, CHUNK)
    w2 = jnp.concatenate(
        [edge_attr.astype(f32), jnp.zeros((pad,), f32)]
    ).reshape(EDGE_ROWS, CHUNK)
    xt = jnp.pad(x.T, ((0, 0), (0, N_PAD - N_NODES)))
    batch_p = jnp.pad(batch.astype(jnp.int32), (0, N_PAD - N_NODES),
                      constant_values=N_GRAPHS).reshape(1, N_PAD)
    zeros_row = jnp.zeros((1, N_PAD), f32)

    mesh = plsc.VectorSubcoreMesh(core_axis_name="c", subcore_axis_name="s")
    sc_params = pltpu.CompilerParams(needs_layout_passes=False)

    agg1t = _sc_agg(xt, src2, dst2, w2, zeros_row, mesh, sc_params)

    h1t = pl.pallas_call(
        _mid_body,
        out_shape=jax.ShapeDtypeStruct((D, N_PAD), f32),
    )(agg1t, xt, W_rel1, W_root1, b_rel1.reshape(D, 1))

    agg3t = _sc_agg(h1t, src2, dst2, w2, zeros_row, mesh, sc_params)

    pooled = pl.pallas_call(
        _post_body,
        out_shape=jax.ShapeDtypeStruct((8, N_GRAPHS), f32),
    )(agg3t, h1t, W_rel3, W_root3, b_rel3.reshape(D, 1), batch_p,
      W_lin, b_lin.reshape(1, 1))

    return pooled[0].reshape(N_GRAPHS, 1)


# speedup vs baseline: 1.4246x; 1.0270x over previous
import jax
import jax.numpy as jnp
from jax import lax
from jax.experimental import pallas as pl
from jax.experimental.pallas import tpu as pltpu
from jax.experimental.pallas import tpu_sc as plsc

N_NODES = 10000
N_EDGES = 320000
D = 128
N_GRAPHS = 64

NC = 2
NS = 16
N_TILES = NC * NS
CHUNK = 128
E_PAD = 327680
EDGE_ROWS = E_PAD // CHUNK
N_PAD = 10112
FPT = D // N_TILES
EBLK = 40
NBLK = EDGE_ROWS // EBLK

_HI = lax.Precision.HIGHEST


def _sc_agg_body(vt_hbm, src_hbm, dst_hbm, w_hbm, zero_hbm, out_hbm,
                 idx_v, ia_v, ib_v,
                 yt0, yt1, yt2, yt3, at0, at1, at2, at3,
                 sa_v, da_v, wa_v, sb_v, db_v, wb_v,
                 sem, sem_a, sem_b):
    cid = lax.axis_index("c")
    sid = lax.axis_index("s")
    wid = cid * NS + sid
    yts = [yt0, yt1, yt2, yt3]
    ats = [at0, at1, at2, at3]
    for f in range(FPT):
        idx_v[...] = lax.iota(jnp.int32, 16) + (wid * FPT + f)
        pltpu.async_copy(vt_hbm.at[idx_v.at[pl.ds(0, 1)]], yts[f], sem).wait()
        pltpu.sync_copy(zero_hbm, ats[f])

    def issue(bi, i_ref, s_ref, d_ref, w_ref, s_sem):
        for t in range((EBLK + 15) // 16):
            i_ref[pl.ds(t * 16, 16)] = jnp.minimum(
                lax.iota(jnp.int32, 16) + (bi * EBLK + t * 16), EDGE_ROWS - 1)
        blk = i_ref.at[pl.ds(0, EBLK)]
        pltpu.async_copy(src_hbm.at[blk], s_ref, s_sem)
        pltpu.async_copy(dst_hbm.at[blk], d_ref, s_sem)
        pltpu.async_copy(w_hbm.at[blk], w_ref, s_sem)

    def wait3(s_ref, d_ref, w_ref, s_sem):
        blk = ia_v.at[pl.ds(0, EBLK)]
        pltpu.make_async_copy(src_hbm.at[blk], s_ref, s_sem).wait()
        pltpu.make_async_copy(dst_hbm.at[blk], d_ref, s_sem).wait()
        pltpu.make_async_copy(w_hbm.at[blk], w_ref, s_sem).wait()

    def compute(s_ref, d_ref, w_ref):
        @plsc.parallel_loop(0, EBLK, 1, unroll=4)
        def row_body(r):
            for j in range(CHUNK // 16):
                s16 = s_ref[r, pl.ds(j * 16, 16)]
                d16 = d_ref[r, pl.ds(j * 16, 16)]
                w16 = w_ref[r, pl.ds(j * 16, 16)]
                for f in range(FPT):
                    v = plsc.load_gather(yts[f].at[0], [s16])
                    plsc.addupdate_scatter(ats[f].at[0], [d16], v * w16)

    issue(0, ia_v, sa_v, da_v, wa_v, sem_a)

    def pair_body(k, carry):
        ba = 2 * k
        wait3(sa_v, da_v, wa_v, sem_a)
        issue(ba + 1, ib_v, sb_v, db_v, wb_v, sem_b)
        compute(sa_v, da_v, wa_v)
        wait3(sb_v, db_v, wb_v, sem_b)

        @pl.when(k < NBLK // 2 - 1)
        def _():
            issue(ba + 2, ia_v, sa_v, da_v, wa_v, sem_a)

        compute(sb_v, db_v, wb_v)
        return carry

    lax.fori_loop(0, NBLK // 2, pair_body, 0)
    for f in range(FPT):
        pltpu.sync_copy(ats[f], out_hbm.at[wid * FPT + f])


def _sc_agg(vt, src2, dst2, w2, zeros_row, mesh, sc_params):
    out = pl.kernel(
        _sc_agg_body,
        out_type=jax.ShapeDtypeStruct((D, 1, N_PAD), jnp.float32),
        mesh=mesh,
        compiler_params=sc_params,
        scratch_types=[
            pltpu.VMEM((16,), jnp.int32),
            pltpu.VMEM((((EBLK + 15) // 16) * 16,), jnp.int32),
            pltpu.VMEM((((EBLK + 15) // 16) * 16,), jnp.int32),
            pltpu.VMEM((1, N_PAD), jnp.float32),
            pltpu.VMEM((1, N_PAD), jnp.float32),
            pltpu.VMEM((1, N_PAD), jnp.float32),
            pltpu.VMEM((1, N_PAD), jnp.float32),
            pltpu.VMEM((1, N_PAD), jnp.float32),
            pltpu.VMEM((1, N_PAD), jnp.float32),
            pltpu.VMEM((1, N_PAD), jnp.float32),
            pltpu.VMEM((1, N_PAD), jnp.float32),
            pltpu.VMEM((EBLK, CHUNK), jnp.int32),
            pltpu.VMEM((EBLK, CHUNK), jnp.int32),
            pltpu.VMEM((EBLK, CHUNK), jnp.float32),
            pltpu.VMEM((EBLK, CHUNK), jnp.int32),
            pltpu.VMEM((EBLK, CHUNK), jnp.int32),
            pltpu.VMEM((EBLK, CHUNK), jnp.float32),
            pltpu.SemaphoreType.DMA,
            pltpu.SemaphoreType.DMA,
            pltpu.SemaphoreType.DMA,
        ],
    )(vt, src2, dst2, w2, zeros_row)
    return out.reshape(D, N_PAD)


def _mid_body(agg_ref, xt_ref, wr1_ref, wt1_ref, br1_ref, out_ref):
    dn = (((1,), (0,)), ((), ()))
    pre = lax.dot_general(wr1_ref[...], agg_ref[...], dn,
                          preferred_element_type=jnp.float32) \
        + lax.dot_general(wt1_ref[...], xt_ref[...], dn,
                          preferred_element_type=jnp.float32) \
        + br1_ref[...]
    out_ref[...] = jnp.maximum(pre, 0.0)


def _post_body(agg_ref, ht_ref, wr3_ref, wt3_ref, br3_ref, batch_ref,
               wlin_ref, blin_ref, out_ref):
    dn = (((1,), (0,)), ((), ()))
    h2 = lax.dot_general(wr3_ref[...], agg_ref[...], dn,
                         preferred_element_type=jnp.float32) \
        + lax.dot_general(wt3_ref[...], ht_ref[...], dn,
                          preferred_element_type=jnp.float32) \
        + br3_ref[...]
    b = batch_ref[...]
    gids = lax.broadcasted_iota(jnp.int32, (N_GRAPHS, N_PAD), 0)
    oh = jnp.where(gids == b, 1.0, 0.0).astype(jnp.float32)
    dn1 = (((1,), (1,)), ((), ()))
    sums = lax.dot_general(h2, oh, dn1, preferred_element_type=jnp.float32,
                           precision=_HI)
    counts = lax.dot_general(jnp.ones((1, N_PAD), jnp.float32), oh, dn1,
                             preferred_element_type=jnp.float32,
                             precision=_HI)
    g = sums / jnp.maximum(counts, 1.0)
    res = lax.dot_general(wlin_ref[...], g, dn,
                          preferred_element_type=jnp.float32) + blin_ref[...]
    out_ref[...] = jnp.broadcast_to(jnp.maximum(res, 0.0), (8, N_GRAPHS))


def kernel(x, edge_index, batch, edge_attr, W_rel1, b_rel1, W_root1,
           W_rel3, b_rel3, W_root3, W_lin, b_lin):
    f32 = jnp.float32
    pad = E_PAD - N_EDGES
    src2 = jnp.concatenate(
        [edge_index[0].astype(jnp.int32), jnp.zeros((pad,), jnp.int32)]
    ).reshape(EDGE_ROWS, CHUNK)
    dst2 = jnp.concatenate(
        [edge_index[1].astype(jnp.int32), jnp.zeros((pad,), jnp.int32)]
    ).reshape(EDGE_ROWS, CHUNK)
    w2 = jnp.concatenate(
        [edge_attr.astype(f32), jnp.zeros((pad,), f32)]
    ).reshape(EDGE_ROWS, CHUNK)
    xt = jnp.pad(x.T, ((0, 0), (0, N_PAD - N_NODES)))
    batch_p = jnp.pad(batch.astype(jnp.int32), (0, N_PAD - N_NODES),
                      constant_values=N_GRAPHS).reshape(1, N_PAD)
    zeros_row = jnp.zeros((1, N_PAD), f32)

    mesh = plsc.VectorSubcoreMesh(core_axis_name="c", subcore_axis_name="s")
    sc_params = pltpu.CompilerParams(needs_layout_passes=False)

    agg1t = _sc_agg(xt, src2, dst2, w2, zeros_row, mesh, sc_params)

    h1t = pl.pallas_call(
        _mid_body,
        out_shape=jax.ShapeDtypeStruct((D, N_PAD), f32),
    )(agg1t, xt, W_rel1, W_root1, b_rel1.reshape(D, 1))

    agg3t = _sc_agg(h1t, src2, dst2, w2, zeros_row, mesh, sc_params)

    pooled = pl.pallas_call(
        _post_body,
        out_shape=jax.ShapeDtypeStruct((8, N_GRAPHS), f32),
    )(agg3t, h1t, W_rel3, W_root3, b_rel3.reshape(D, 1), batch_p,
      W_lin, b_lin.reshape(1, 1))

    return pooled[0].reshape(N_GRAPHS, 1)
